# Initial kernel scaffold; baseline (speedup 1.0000x reference)
import jax
import jax.numpy as jnp
from jax.experimental import pallas as pl


def kernel(z_user, z_movie, rel_emb, edge_label_index):
    raise NotImplementedError("write your pallas kernel here")


# speedup vs baseline: 1.0104x; 1.0104x over previous
import functools

import jax
import jax.numpy as jnp
from jax import lax
from jax.experimental import pallas as pl
from jax.experimental.pallas import tpu as pltpu
from jax.experimental.pallas import tpu_sc as plsc

H = 128
NUM_REL = 5
NC = 2
NS = 16
NW = NC * NS
CB = 80
DC = 16


def _decode(z_user, z_movie, rel_emb, src_idx, dst_idx):
    E = src_idx.shape[0]
    assert E % NW == 0
    epw = E // NW
    assert epw % CB == 0
    nchunk = epw // CB
    nsub = CB // 16
    mesh = plsc.VectorSubcoreMesh(core_axis_name="c", subcore_axis_name="s",
                                  num_cores=NC, num_subcores=NS)

    @functools.partial(
        pl.kernel,
        out_type=jax.ShapeDtypeStruct((E,), jnp.float32),
        mesh=mesh,
        compiler_params=pltpu.CompilerParams(needs_layout_passes=False),
        scratch_types=[
            pltpu.VMEM((NUM_REL, H), jnp.float32),
            pltpu.VMEM((CB,), jnp.int32),
            pltpu.VMEM((CB,), jnp.int32),
            pltpu.VMEM((CB, H), jnp.float32),
            pltpu.VMEM((CB, H), jnp.float32),
            pltpu.VMEM((CB,), jnp.float32),
            pltpu.SemaphoreType.DMA,
            pltpu.SemaphoreType.DMA,
        ],
    )
    def decode_k(zu, zm, rel, src, dst, out, rel_v, si_v, di_v,
                 u_v, m_v, o_v, sem1, sem2):
        wid = lax.axis_index("s") * NC + lax.axis_index("c")
        base_w = wid * epw
        pltpu.sync_copy(rel, rel_v)
        iota = lax.iota(jnp.int32, 16)
        row_idx = [iota + (g * 16) for g in range(nsub)]

        @pl.loop(0, nchunk)
        def _chunk(c):
            base = base_w + c * CB
            pltpu.sync_copy(src.at[pl.ds(base, CB)], si_v)
            pltpu.sync_copy(dst.at[pl.ds(base, CB)], di_v)
            g1 = pltpu.async_copy(zu.at[si_v], u_v, sem1)
            g2 = pltpu.async_copy(zm.at[di_v], m_v, sem2)
            g1.wait()
            g2.wait()

            zero = jnp.zeros((16,), jnp.float32)
            init = tuple(tuple(zero for _ in range(NUM_REL))
                         for _ in range(nsub))

            @pl.loop(0, H // DC, init_carry=init)
            def acc_loop(dc, acc):
                acc = [list(a) for a in acc]
                col_base = jnp.full((16,), dc * DC, jnp.int32)
                rv = [rel_v[l, pl.ds(dc * DC, DC)] for l in range(NUM_REL)]
                for j in range(DC):
                    col = col_base + j
                    lane = jnp.full((16,), j, jnp.int32)
                    rb = [rv[l].at[lane].get(mode="promise_in_bounds")
                          for l in range(NUM_REL)]
                    for g in range(nsub):
                        u = plsc.load_gather(u_v, [row_idx[g], col])
                        m = plsc.load_gather(m_v, [row_idx[g], col])
                        s = u * m
                        for l in range(NUM_REL):
                            acc[g][l] = acc[g][l] + s * rb[l]
                return tuple(tuple(a) for a in acc)

            for g in range(nsub):
                a = acc_loop[g]
                mx = a[0]
                for l in range(1, NUM_REL):
                    mx = jnp.maximum(mx, a[l])
                e = [jnp.exp(a[l] - mx) for l in range(NUM_REL)]
                den = e[0]
                num = zero
                for l in range(1, NUM_REL):
                    den = den + e[l]
                    num = num + jnp.float32(l) * e[l]
                o_v[pl.ds(g * 16, 16)] = num / den
            pltpu.sync_copy(o_v, out.at[pl.ds(base, CB)])

    return decode_k(z_user, z_movie, rel_emb, src_idx, dst_idx)


def kernel(z_user, z_movie, rel_emb, edge_label_index):
    src = edge_label_index[0]
    dst = edge_label_index[1]
    return _decode(z_user, z_movie, rel_emb, src, dst)


# speedup vs baseline: 1.1780x; 1.1659x over previous
import functools

import jax
import jax.numpy as jnp
from jax import lax
from jax.experimental import pallas as pl
from jax.experimental.pallas import tpu as pltpu
from jax.experimental.pallas import tpu_sc as plsc

H = 128
NUM_REL = 5
NC = 2
NS = 16
NW = NC * NS
CB = 80
DC = 16
NSUB = CB // 16


def _decode(z_user, z_movie, rel_emb, src3, dst3):
    nw, npw, cb = src3.shape
    assert cb == CB and nw == NW and npw % 2 == 1
    npairs = npw // 2
    mesh = plsc.VectorSubcoreMesh(core_axis_name="c", subcore_axis_name="s",
                                  num_cores=NC, num_subcores=NS)

    @functools.partial(
        pl.kernel,
        out_type=jax.ShapeDtypeStruct((NW, npw, CB), jnp.float32),
        mesh=mesh,
        compiler_params=pltpu.CompilerParams(needs_layout_passes=False),
        scratch_types=[
            pltpu.VMEM((NUM_REL, H), jnp.float32),
            pltpu.VMEM((npw, CB), jnp.int32),
            pltpu.VMEM((npw, CB), jnp.int32),
            pltpu.VMEM((CB, H), jnp.float32),
            pltpu.VMEM((CB, H), jnp.float32),
            pltpu.VMEM((CB, H), jnp.float32),
            pltpu.VMEM((CB, H), jnp.float32),
            pltpu.VMEM((npw, CB), jnp.float32),
            pltpu.SemaphoreType.DMA,
            pltpu.SemaphoreType.DMA,
        ],
    )
    def decode_k(zu, zm, rel, src, dst, out, rel_v, si_v, di_v,
                 uA, mA, uB, mB, o_v, semA, semB):
        wid = lax.axis_index("s") * NC + lax.axis_index("c")
        pltpu.sync_copy(rel, rel_v)
        pltpu.sync_copy(src.at[wid], si_v)
        pltpu.sync_copy(dst.at[wid], di_v)

        iota = lax.iota(jnp.int32, 16)
        row_idx = [iota + (g * 16) for g in range(NSUB)]
        zero = jnp.zeros((16,), jnp.float32)

        def issue(c, u_buf, m_buf, sem):
            pltpu.async_copy(zu.at[si_v.at[c]], u_buf, sem)
            pltpu.async_copy(zm.at[di_v.at[c]], m_buf, sem)

        def drain(c, u_buf, m_buf, sem):
            pltpu.make_async_copy(zu.at[si_v.at[c]], u_buf, sem).wait()
            pltpu.make_async_copy(zm.at[di_v.at[c]], m_buf, sem).wait()

        def compute(c, u_buf, m_buf):
            init = tuple(tuple(zero for _ in range(NUM_REL))
                         for _ in range(NSUB))

            @pl.loop(0, H // DC, init_carry=init)
            def acc_loop(dc, acc):
                acc = [list(a) for a in acc]
                col_base = jnp.full((16,), dc * DC, jnp.int32)
                rv = [rel_v[l, pl.ds(dc * DC, DC)] for l in range(NUM_REL)]
                for j in range(DC):
                    col = col_base + j
                    lane = jnp.full((16,), j, jnp.int32)
                    rb = [rv[l].at[lane].get(mode="promise_in_bounds")
                          for l in range(NUM_REL)]
                    for g in range(NSUB):
                        u = plsc.load_gather(u_buf, [row_idx[g], col])
                        m = plsc.load_gather(m_buf, [row_idx[g], col])
                        s = u * m
                        for l in range(NUM_REL):
                            acc[g][l] = acc[g][l] + s * rb[l]
                return tuple(tuple(a) for a in acc)

            for g in range(NSUB):
                a = acc_loop[g]
                mx = a[0]
                for l in range(1, NUM_REL):
                    mx = jnp.maximum(mx, a[l])
                e = [jnp.exp(a[l] - mx) for l in range(NUM_REL)]
                den = e[0]
                num = zero
                for l in range(1, NUM_REL):
                    den = den + e[l]
                    num = num + jnp.float32(l) * e[l]
                o_v[c, pl.ds(g * 16, 16)] = num / den

        issue(0, uA, mA, semA)

        @pl.loop(0, npairs)
        def _pair(i):
            cA = 2 * i
            cB = cA + 1
            issue(cB, uB, mB, semB)
            drain(cA, uA, mA, semA)
            compute(cA, uA, mA)
            issue(cA + 2, uA, mA, semA)
            drain(cB, uB, mB, semB)
            compute(cB, uB, mB)

        last = npw - 1
        drain(last, uA, mA, semA)
        compute(last, uA, mA)

        pltpu.sync_copy(o_v, out.at[wid])

    return decode_k(z_user, z_movie, rel_emb, src3, dst3)


def kernel(z_user, z_movie, rel_emb, edge_label_index):
    E = edge_label_index.shape[1]
    npw = E // (NW * CB)
    src3 = edge_label_index[0].reshape(NW, npw, CB)
    dst3 = edge_label_index[1].reshape(NW, npw, CB)
    out3 = _decode(z_user, z_movie, rel_emb, src3, dst3)
    return out3.reshape(E)


# speedup vs baseline: 5.8772x; 4.9890x over previous
import functools

import jax
import jax.numpy as jnp
from jax import lax
from jax.experimental import pallas as pl
from jax.experimental.pallas import tpu as pltpu
from jax.experimental.pallas import tpu_sc as plsc

H = 128
NUM_REL = 5
NC = 2
NS = 16
NW = NC * NS
CB = 80
DC = 16
NSUB = CB // 16


def _decode(z_user, z_movie, rel_emb, src3, dst3):
    nw, npw, cb = src3.shape
    assert cb == CB and nw == NW and npw % 2 == 1
    npairs = npw // 2
    mesh = plsc.VectorSubcoreMesh(core_axis_name="c", subcore_axis_name="s",
                                  num_cores=NC, num_subcores=NS)

    @functools.partial(
        pl.kernel,
        out_type=jax.ShapeDtypeStruct((NW, npw, CB), jnp.float32),
        mesh=mesh,
        compiler_params=pltpu.CompilerParams(needs_layout_passes=False),
        scratch_types=[
            pltpu.VMEM((NUM_REL, H), jnp.float32),
            pltpu.VMEM((npw, CB), jnp.int32),
            pltpu.VMEM((npw, CB), jnp.int32),
            pltpu.VMEM((CB, H), jnp.float32),
            pltpu.VMEM((CB, H), jnp.float32),
            pltpu.VMEM((CB, H), jnp.float32),
            pltpu.VMEM((CB, H), jnp.float32),
            pltpu.VMEM((npw, CB), jnp.float32),
            pltpu.SemaphoreType.DMA,
            pltpu.SemaphoreType.DMA,
        ],
    )
    def decode_k(zu, zm, rel, src, dst, out, rel_v, si_v, di_v,
                 uA, mA, uB, mB, o_v, semA, semB):
        wid = lax.axis_index("s") * NC + lax.axis_index("c")
        pltpu.sync_copy(rel, rel_v)
        pltpu.sync_copy(src.at[wid], si_v)
        pltpu.sync_copy(dst.at[wid], di_v)

        iota = lax.iota(jnp.int32, 16)
        row_idx = [iota + (g * 16) for g in range(NSUB)]
        zero = jnp.zeros((16,), jnp.float32)

        def issue(c, u_buf, m_buf, sem):
            pltpu.async_copy(zu.at[si_v.at[c]], u_buf, sem)
            pltpu.async_copy(zm.at[di_v.at[c]], m_buf, sem)

        def drain(c, u_buf, m_buf, sem):
            pltpu.make_async_copy(zu.at[si_v.at[c]], u_buf, sem).wait()
            pltpu.make_async_copy(zm.at[di_v.at[c]], m_buf, sem).wait()

        def compute(c, u_buf, m_buf):
            init = tuple(tuple(zero for _ in range(NUM_REL))
                         for _ in range(NSUB))

            @pl.loop(0, H // DC, init_carry=init)
            def acc_loop(dc, acc):
                acc = [list(a) for a in acc]
                col_base = jnp.full((16,), dc * DC, jnp.int32)
                rv = [rel_v[l, pl.ds(dc * DC, DC)] for l in range(NUM_REL)]
                for j in range(DC):
                    shift = (j + iota) % DC
                    col = col_base + shift
                    rb = [rv[l].at[shift].get(mode="promise_in_bounds")
                          for l in range(NUM_REL)]
                    for g in range(NSUB):
                        u = plsc.load_gather(u_buf, [row_idx[g], col])
                        m = plsc.load_gather(m_buf, [row_idx[g], col])
                        s = u * m
                        for l in range(NUM_REL):
                            acc[g][l] = acc[g][l] + s * rb[l]
                return tuple(tuple(a) for a in acc)

            for g in range(NSUB):
                a = acc_loop[g]
                mx = a[0]
                for l in range(1, NUM_REL):
                    mx = jnp.maximum(mx, a[l])
                e = [jnp.exp(a[l] - mx) for l in range(NUM_REL)]
                den = e[0]
                num = zero
                for l in range(1, NUM_REL):
                    den = den + e[l]
                    num = num + jnp.float32(l) * e[l]
                o_v[c, pl.ds(g * 16, 16)] = num / den

        issue(0, uA, mA, semA)

        @pl.loop(0, npairs)
        def _pair(i):
            cA = 2 * i
            cB = cA + 1
            issue(cB, uB, mB, semB)
            drain(cA, uA, mA, semA)
            compute(cA, uA, mA)
            issue(cA + 2, uA, mA, semA)
            drain(cB, uB, mB, semB)
            compute(cB, uB, mB)

        last = npw - 1
        drain(last, uA, mA, semA)
        compute(last, uA, mA)

        pltpu.sync_copy(o_v, out.at[wid])

    return decode_k(z_user, z_movie, rel_emb, src3, dst3)


def kernel(z_user, z_movie, rel_emb, edge_label_index):
    E = edge_label_index.shape[1]
    npw = E // (NW * CB)
    src3 = edge_label_index[0].reshape(NW, npw, CB)
    dst3 = edge_label_index[1].reshape(NW, npw, CB)
    out3 = _decode(z_user, z_movie, rel_emb, src3, dst3)
    return out3.reshape(E)


# speedup vs baseline: 6.3976x; 1.0885x over previous
import functools

import jax
import jax.numpy as jnp
from jax import lax
from jax.experimental import pallas as pl
from jax.experimental.pallas import tpu as pltpu
from jax.experimental.pallas import tpu_sc as plsc

H = 128
NUM_REL = 5
NC = 2
NS = 16
NW = NC * NS
CB = 80
DC = 16
NSUB = CB // 16


def _decode(z_user, z_movie, rel_emb, src3, dst3):
    nw, npw, cb = src3.shape
    assert cb == CB and nw == NW and npw % 2 == 1
    npairs = npw // 2
    mesh = plsc.VectorSubcoreMesh(core_axis_name="c", subcore_axis_name="s",
                                  num_cores=NC, num_subcores=NS)

    @functools.partial(
        pl.kernel,
        out_type=jax.ShapeDtypeStruct((NW, npw, CB), jnp.float32),
        mesh=mesh,
        compiler_params=pltpu.CompilerParams(needs_layout_passes=False),
        scratch_types=[
            pltpu.VMEM((NUM_REL - 1, H), jnp.float32),
            pltpu.VMEM((npw, CB), jnp.int32),
            pltpu.VMEM((npw, CB), jnp.int32),
            pltpu.VMEM((CB, H), jnp.float32),
            pltpu.VMEM((CB, H), jnp.float32),
            pltpu.VMEM((CB, H), jnp.float32),
            pltpu.VMEM((CB, H), jnp.float32),
            pltpu.VMEM((npw, CB), jnp.float32),
            pltpu.SemaphoreType.DMA,
            pltpu.SemaphoreType.DMA,
        ],
    )
    def decode_k(zu, zm, rel, src, dst, out, rel_v, si_v, di_v,
                 uA, mA, uB, mB, o_v, semA, semB):
        wid = lax.axis_index("s") * NC + lax.axis_index("c")
        pltpu.sync_copy(rel, rel_v)
        pltpu.sync_copy(src.at[wid], si_v)
        pltpu.sync_copy(dst.at[wid], di_v)

        iota = lax.iota(jnp.int32, 16)
        row_idx = [iota + (g * 16) for g in range(NSUB)]
        zero = jnp.zeros((16,), jnp.float32)

        def issue(c, u_buf, m_buf, sem):
            pltpu.async_copy(zu.at[si_v.at[c]], u_buf, sem)
            pltpu.async_copy(zm.at[di_v.at[c]], m_buf, sem)

        def drain(c, u_buf, m_buf, sem):
            pltpu.make_async_copy(zu.at[si_v.at[c]], u_buf, sem).wait()
            pltpu.make_async_copy(zm.at[di_v.at[c]], m_buf, sem).wait()

        def compute(c, u_buf, m_buf):
            init = tuple(tuple(zero for _ in range(NUM_REL - 1))
                         for _ in range(NSUB))

            @pl.loop(0, H // DC, init_carry=init)
            def acc_loop(dc, acc):
                acc = [list(a) for a in acc]
                col_base = jnp.full((16,), dc * DC, jnp.int32)
                rv = [rel_v[l, pl.ds(dc * DC, DC)]
                      for l in range(NUM_REL - 1)]
                for j in range(DC):
                    shift = (j + iota) % DC
                    col = col_base + shift
                    rb = [rv[l].at[shift].get(mode="promise_in_bounds")
                          for l in range(NUM_REL - 1)]
                    for g in range(NSUB):
                        u = plsc.load_gather(u_buf, [row_idx[g], col])
                        m = plsc.load_gather(m_buf, [row_idx[g], col])
                        s = u * m
                        for l in range(NUM_REL - 1):
                            acc[g][l] = acc[g][l] + s * rb[l]
                return tuple(tuple(a) for a in acc)

            for g in range(NSUB):
                a = acc_loop[g]
                mx = zero
                for l in range(NUM_REL - 1):
                    mx = jnp.maximum(mx, a[l])
                e0 = jnp.exp(zero - mx)
                e = [jnp.exp(a[l] - mx) for l in range(NUM_REL - 1)]
                den = e0
                num = zero
                for l in range(NUM_REL - 1):
                    den = den + e[l]
                    num = num + jnp.float32(l + 1) * e[l]
                o_v[c, pl.ds(g * 16, 16)] = num / den

        issue(0, uA, mA, semA)

        @pl.loop(0, npairs)
        def _pair(i):
            cA = 2 * i
            cB = cA + 1
            issue(cB, uB, mB, semB)
            drain(cA, uA, mA, semA)
            compute(cA, uA, mA)
            issue(cA + 2, uA, mA, semA)
            drain(cB, uB, mB, semB)
            compute(cB, uB, mB)

        last = npw - 1
        drain(last, uA, mA, semA)
        compute(last, uA, mA)

        pltpu.sync_copy(o_v, out.at[wid])

    return decode_k(z_user, z_movie, rel_emb, src3, dst3)


def kernel(z_user, z_movie, rel_emb, edge_label_index):
    E = edge_label_index.shape[1]
    npw = E // (NW * CB)
    src3 = edge_label_index[0].reshape(NW, npw, CB)
    dst3 = edge_label_index[1].reshape(NW, npw, CB)
    rel_shift = rel_emb[1:] - rel_emb[0:1]
    out3 = _decode(z_user, z_movie, rel_shift, src3, dst3)
    return out3.reshape(E)
